# baseline (speedup 1.0000x reference)
import jax
import jax.numpy as jnp
from jax.experimental import pallas as pl

_RATIO = 0.5
_R = 0.2
_K = 32


def _copy_kernel(a_ref, o_ref):
    o_ref[...] = a_ref[...]


def _fps(pos, n_samples):
    N = pos.shape[0]

    def body(i, state):
        idxs, dists = state
        last = idxs[i - 1]
        d = jnp.sum((pos - pos[last]) ** 2, axis=-1)
        dists = jnp.minimum(dists, d)
        idxs = idxs.at[i].set(jnp.argmax(dists).astype(jnp.int32))
        return (idxs, dists)

    idxs = jnp.zeros((n_samples,), dtype=jnp.int32)
    dists = jnp.full((N,), jnp.inf, dtype=pos.dtype)
    idxs, _ = jax.lax.fori_loop(1, n_samples, body, (idxs, dists))
    return idxs


def kernel(x, pos, batch, W1, b1, W2, b2):
    pos = pl.pallas_call(
        _copy_kernel, out_shape=jax.ShapeDtypeStruct(pos.shape, pos.dtype)
    )(pos)
    N = pos.shape[0]
    M = int(N * _RATIO)
    idx = _fps(jax.lax.stop_gradient(pos), M)
    pos_q = jnp.take(pos, idx, axis=0)
    p = jax.lax.stop_gradient(pos)
    q = jax.lax.stop_gradient(pos_q)
    d2 = jnp.sum(q * q, axis=1)[:, None] + jnp.sum(p * p, axis=1)[None, :] - 2.0 * (q @ p.T)
    masked = jnp.where(d2 <= _R * _R, -d2, -jnp.inf)
    vals, cols = jax.lax.top_k(masked, _K)
    valid = vals > -jnp.inf
    rows = jnp.broadcast_to(jnp.arange(M, dtype=jnp.int32)[:, None], (M, _K))
    col_f = cols.reshape(-1)
    row_f = rows.reshape(-1)
    x_j = jnp.take(x, col_f, axis=0)
    pos_j = jnp.take(pos, col_f, axis=0)
    pos_i = jnp.take(pos_q, row_f, axis=0)
    msg_in = jnp.concatenate([x_j, pos_j - pos_i], axis=-1)
    h = jnp.maximum(msg_in @ W1 + b1, 0.0)
    h = jnp.maximum(h @ W2 + b2, 0.0)
    msg = jnp.where(valid.reshape(-1)[:, None], h, -jnp.inf)
    out = jnp.max(msg.reshape(M, _K, -1), axis=1)
    batch_out = jnp.take(batch, idx, axis=0)
    return (out, pos_q, batch_out)


# speedup vs baseline: 5.4767x; 5.4767x over previous
import functools

import jax
import jax.numpy as jnp
from jax.experimental import pallas as pl
from jax.experimental.pallas import tpu as pltpu

_RATIO = 0.5
_R = 0.2
_K = 32
_N = 10000
_NPAD = 10240
_ROWS = 8
_COLS = _NPAD // _ROWS
_M = int(_N * _RATIO)
_MPAD = 5120
_BC = 64
_E = _BC * _K
_BQ = 8


def _fps_kernel(px_ref, py_ref, pz_ref, idx_ref, *, n_samples):
    px = px_ref[...]
    py = py_ref[...]
    pz = pz_ref[...]
    n = (jax.lax.broadcasted_iota(jnp.int32, (_ROWS, _COLS), 0) * _COLS
         + jax.lax.broadcasted_iota(jnp.int32, (_ROWS, _COLS), 1))
    pad = n >= _N
    neg = jnp.float32(-jnp.inf)
    bigi = jnp.int32(2**30)

    idx_ref[0] = jnp.int32(0)
    sel0 = n == 0
    lx0 = jnp.sum(jnp.where(sel0, px, 0.0))
    ly0 = jnp.sum(jnp.where(sel0, py, 0.0))
    lz0 = jnp.sum(jnp.where(sel0, pz, 0.0))
    dists0 = jnp.where(pad, neg, jnp.float32(jnp.inf))

    def body(i, carry):
        lx, ly, lz, dists = carry
        dx = px - lx
        dy = py - ly
        dz = pz - lz
        d = (dx * dx + dz * dz) + dy * dy
        dists = jnp.minimum(dists, d)
        dists = jnp.where(pad, neg, dists)
        m = jnp.max(dists)
        j = jnp.min(jnp.where(dists == m, n, bigi))
        idx_ref[i] = j
        sel = n == j
        lx = jnp.sum(jnp.where(sel, px, 0.0))
        ly = jnp.sum(jnp.where(sel, py, 0.0))
        lz = jnp.sum(jnp.where(sel, pz, 0.0))
        return (lx, ly, lz, dists)

    jax.lax.fori_loop(1, n_samples, body, (lx0, ly0, lz0, dists0))


def _select_kernel(q_ref, pt_ref, cols_ref, valid_ref):
    qv = q_ref[...]
    qpad = jnp.concatenate([qv, jnp.zeros((_BQ, 5), jnp.float32)], axis=1)
    pt = pt_ref[...]
    px = pt[0:1, :]
    py = pt[1:2, :]
    pz = pt[2:3, :]
    p2 = px * px + py * py + pz * pz
    qx = qv[:, 0:1]
    qy = qv[:, 1:2]
    qz = qv[:, 2:3]
    q2 = qx * qx + qy * qy + qz * qz
    qp = jnp.dot(qpad, pt, preferred_element_type=jnp.float32)
    d2 = q2 + p2 - 2.0 * qp
    r2 = jnp.float32(_R * _R)
    inf = jnp.float32(jnp.inf)
    bigi = jnp.int32(2**30)
    coln = jax.lax.broadcasted_iota(jnp.int32, (_BQ, _NPAD), 1)
    d2m = jnp.where(d2 <= r2, d2, inf)
    for k in range(_K):
        m = jnp.min(d2m, axis=1, keepdims=True)
        jv = jnp.min(jnp.where(d2m == m, coln, bigi), axis=1, keepdims=True)
        vk = m < inf
        cols_ref[:, k : k + 1] = jnp.where(vk, jv, 0)
        valid_ref[:, k : k + 1] = jnp.where(vk, 1.0, 0.0).astype(jnp.float32)
        d2m = jnp.where(coln == jv, inf, d2m)


def _mlp_kernel(colsr_ref, vr_ref, xcat_ref, q_ref, w1_ref, w1p_ref, b1_ref,
                w2_ref, b2_ref, o_ref, feat_ref):
    def gather_body(t, _):
        col = colsr_ref[t]
        feat_ref[pl.ds(t, 1), :] = xcat_ref[pl.ds(col, 1), :]
        return 0

    jax.lax.fori_loop(0, _E, gather_body, 0)
    f = feat_ref[...]
    h1 = jnp.dot(f, w1_ref[...], preferred_element_type=jnp.float32)
    corr = jnp.dot(q_ref[...], w1p_ref[...], preferred_element_type=jnp.float32)
    corrfull = jnp.concatenate([corr] * _K, axis=0)
    h1 = jnp.maximum(h1 - corrfull + b1_ref[...], 0.0)
    h2 = jnp.dot(h1, w2_ref[...], preferred_element_type=jnp.float32)
    h2 = jnp.maximum(h2 + b2_ref[...], 0.0)
    h2 = jnp.where(vr_ref[...] > 0.5, h2, -jnp.inf)
    acc = h2[0:_BC, :]
    for k in range(1, _K):
        acc = jnp.maximum(acc, h2[k * _BC : (k + 1) * _BC, :])
    o_ref[...] = acc


def kernel(x, pos, batch, W1, b1, W2, b2):
    pos = jax.lax.stop_gradient(pos)
    pn = jnp.pad(pos, ((0, _NPAD - _N), (0, 0)))
    px = pn[:, 0].reshape(_ROWS, _COLS)
    py = pn[:, 1].reshape(_ROWS, _COLS)
    pz = pn[:, 2].reshape(_ROWS, _COLS)
    idx = pl.pallas_call(
        functools.partial(_fps_kernel, n_samples=_M),
        out_shape=jax.ShapeDtypeStruct((_M,), jnp.int32),
        out_specs=pl.BlockSpec(memory_space=pltpu.SMEM),
    )(px, py, pz)
    pos_q = jnp.take(pos, idx, axis=0)

    ptpad = jnp.pad(pos.T, ((0, 5), (0, _NPAD - _N)), constant_values=1e9)
    ptpad = ptpad.at[3:, :].set(0.0)
    cols, validf = pl.pallas_call(
        _select_kernel,
        grid=(_M // _BQ,),
        in_specs=[
            pl.BlockSpec((_BQ, 3), lambda b: (b, 0)),
            pl.BlockSpec((8, _NPAD), lambda b: (0, 0)),
        ],
        out_specs=[
            pl.BlockSpec((_BQ, _K), lambda b: (b, 0)),
            pl.BlockSpec((_BQ, _K), lambda b: (b, 0)),
        ],
        out_shape=[
            jax.ShapeDtypeStruct((_M, _K), jnp.int32),
            jax.ShapeDtypeStruct((_M, _K), jnp.float32),
        ],
    )(pos_q, ptpad)

    nb = _MPAD // _BC
    colsp = jnp.pad(cols, ((0, _MPAD - _M), (0, 0)))
    validp = jnp.pad(validf, ((0, _MPAD - _M), (0, 0)))
    colsr = colsp.reshape(nb, _BC, _K).transpose(0, 2, 1).reshape(-1)
    vr = validp.reshape(nb, _BC, _K).transpose(0, 2, 1).reshape(-1, 1)
    xcat = jnp.concatenate([x, pos, jnp.zeros((_N, 5), jnp.float32)], axis=1)
    qpad = jnp.pad(pos_q, ((0, _MPAD - _M), (0, 5)))
    w1cat = jnp.concatenate([W1, jnp.zeros((5, 128), jnp.float32)], axis=0)
    w1p = jnp.concatenate([W1[128:131], jnp.zeros((5, 128), jnp.float32)], axis=0)
    out_pad = pl.pallas_call(
        _mlp_kernel,
        grid=(nb,),
        in_specs=[
            pl.BlockSpec((_E,), lambda b: (b,), memory_space=pltpu.SMEM),
            pl.BlockSpec((_E, 1), lambda b: (b, 0)),
            pl.BlockSpec((_N, 136), lambda b: (0, 0)),
            pl.BlockSpec((_BC, 8), lambda b: (b, 0)),
            pl.BlockSpec((136, 128), lambda b: (0, 0)),
            pl.BlockSpec((8, 128), lambda b: (0, 0)),
            pl.BlockSpec((1, 128), lambda b: (0, 0)),
            pl.BlockSpec((128, 128), lambda b: (0, 0)),
            pl.BlockSpec((1, 128), lambda b: (0, 0)),
        ],
        out_specs=pl.BlockSpec((_BC, 128), lambda b: (b, 0)),
        out_shape=jax.ShapeDtypeStruct((_MPAD, 128), jnp.float32),
        scratch_shapes=[pltpu.VMEM((_E, 136), jnp.float32)],
    )(colsr, vr, xcat, qpad, w1cat, w1p, b1[None, :], W2, b2[None, :])
    out = out_pad[:_M]
    batch_out = jnp.take(batch, idx, axis=0)
    return (out, pos_q, batch_out)


# speedup vs baseline: 8.4238x; 1.5381x over previous
import functools

import jax
import jax.numpy as jnp
from jax.experimental import pallas as pl
from jax.experimental.pallas import tpu as pltpu

_RATIO = 0.5
_R = 0.2
_K = 32
_N = 10000
_NPAD = 10240
_ROWS = 8
_COLS = _NPAD // _ROWS
_M = int(_N * _RATIO)
_MPAD = 5120
_BC = 64
_E = _BC * _K
_BQ = 32


def _fold(ta, tb):
    va, na, xa, ya, za = ta
    vb, nb, xb, yb, zb = tb
    takeb = (vb > va) | ((vb == va) & (nb < na))
    return (
        jnp.where(takeb, vb, va),
        jnp.where(takeb, nb, na),
        jnp.where(takeb, xb, xa),
        jnp.where(takeb, yb, ya),
        jnp.where(takeb, zb, za),
    )


def _fps_kernel(px_ref, py_ref, pz_ref, idx_ref, *, n_samples):
    px = px_ref[...]
    py = py_ref[...]
    pz = pz_ref[...]
    n = (jax.lax.broadcasted_iota(jnp.int32, (_ROWS, _COLS), 0) * _COLS
         + jax.lax.broadcasted_iota(jnp.int32, (_ROWS, _COLS), 1))
    pad = n >= _N
    neg = jnp.float32(-jnp.inf)

    idx_ref[0] = jnp.int32(0)
    lx0 = px[0:1, 0:1]
    ly0 = py[0:1, 0:1]
    lz0 = pz[0:1, 0:1]
    dists0 = jnp.where(pad, neg, jnp.float32(jnp.inf))

    def argmax_tree(dists):
        t = (dists, n, px, py, pz)
        seg = 256
        s0 = tuple(a[:, 0 * seg : 1 * seg] for a in t)
        s1 = tuple(a[:, 1 * seg : 2 * seg] for a in t)
        s2 = tuple(a[:, 2 * seg : 3 * seg] for a in t)
        s3 = tuple(a[:, 3 * seg : 4 * seg] for a in t)
        s4 = tuple(a[:, 4 * seg : 5 * seg] for a in t)
        t = _fold(_fold(_fold(s0, s1), _fold(s2, s3)), s4)
        w = seg
        while w > 1:
            h = w // 2
            t = _fold(tuple(a[:, :h] for a in t), tuple(a[:, h:] for a in t))
            w = h
        r = 8
        while r > 1:
            h = r // 2
            t = _fold(tuple(a[:h, :] for a in t), tuple(a[h:, :] for a in t))
            r = h
        return t

    def body(i, carry):
        lx, ly, lz, dists = carry
        dx = px - lx
        dy = py - ly
        dz = pz - lz
        d = (dx * dx + dz * dz) + dy * dy
        dists = jnp.minimum(dists, d)
        _, nn, xx, yy, zz = argmax_tree(dists)
        idx_ref[i] = jnp.min(nn)
        return (xx, yy, zz, dists)

    jax.lax.fori_loop(1, n_samples, body, (lx0, ly0, lz0, dists0))


def _select_kernel(q_ref, pt_ref, cols_ref, valid_ref):
    qv = q_ref[...]
    qpad = jnp.concatenate([qv, jnp.zeros((_BQ, 5), jnp.float32)], axis=1)
    pt = pt_ref[...]
    px = pt[0:1, :]
    py = pt[1:2, :]
    pz = pt[2:3, :]
    p2 = px * px + py * py + pz * pz
    qx = qv[:, 0:1]
    qy = qv[:, 1:2]
    qz = qv[:, 2:3]
    q2 = qx * qx + qy * qy + qz * qz
    qp = jnp.dot(qpad, pt, preferred_element_type=jnp.float32)
    d2 = q2 + p2 - 2.0 * qp
    r2 = jnp.float32(_R * _R)
    inf = jnp.float32(jnp.inf)
    bigi = jnp.int32(2**30)
    coln = jax.lax.broadcasted_iota(jnp.int32, (_BQ, _NPAD), 1)
    d2m = jnp.where(d2 <= r2, d2, inf)
    for k in range(_K):
        m = jnp.min(d2m, axis=1, keepdims=True)
        jv = jnp.min(jnp.where(d2m == m, coln, bigi), axis=1, keepdims=True)
        vk = m < inf
        cols_ref[:, k : k + 1] = jnp.where(vk, jv, 0)
        valid_ref[:, k : k + 1] = jnp.where(vk, 1.0, 0.0).astype(jnp.float32)
        d2m = jnp.where(coln == jv, inf, d2m)


def _mlp_kernel(colsr_ref, vr_ref, xcat_ref, q_ref, w1_ref, w1p_ref, b1_ref,
                w2_ref, b2_ref, o_ref, feat_ref):
    def gather_body(t, _):
        col = colsr_ref[t]
        feat_ref[pl.ds(t, 1), :] = xcat_ref[pl.ds(col, 1), :]
        return 0

    jax.lax.fori_loop(0, _E, gather_body, 0)
    f = feat_ref[...]
    h1 = jnp.dot(f, w1_ref[...], preferred_element_type=jnp.float32)
    corr = jnp.dot(q_ref[...], w1p_ref[...], preferred_element_type=jnp.float32)
    corrfull = jnp.concatenate([corr] * _K, axis=0)
    h1 = jnp.maximum(h1 - corrfull + b1_ref[...], 0.0)
    h2 = jnp.dot(h1, w2_ref[...], preferred_element_type=jnp.float32)
    h2 = jnp.maximum(h2 + b2_ref[...], 0.0)
    h2 = jnp.where(vr_ref[...] > 0.5, h2, -jnp.inf)
    acc = h2[0:_BC, :]
    for k in range(1, _K):
        acc = jnp.maximum(acc, h2[k * _BC : (k + 1) * _BC, :])
    o_ref[...] = acc


def kernel(x, pos, batch, W1, b1, W2, b2):
    pos = jax.lax.stop_gradient(pos)
    pn = jnp.pad(pos, ((0, _NPAD - _N), (0, 0)))
    px = pn[:, 0].reshape(_ROWS, _COLS)
    py = pn[:, 1].reshape(_ROWS, _COLS)
    pz = pn[:, 2].reshape(_ROWS, _COLS)
    idx = pl.pallas_call(
        functools.partial(_fps_kernel, n_samples=_M),
        out_shape=jax.ShapeDtypeStruct((_M,), jnp.int32),
        out_specs=pl.BlockSpec(memory_space=pltpu.SMEM),
    )(px, py, pz)
    pos_q = jnp.take(pos, idx, axis=0)

    qpadded = jnp.pad(pos_q, ((0, _MPAD - _M), (0, 0)), constant_values=1e9)
    ptpad = jnp.pad(pos.T, ((0, 5), (0, _NPAD - _N)), constant_values=-1e9)
    ptpad = ptpad.at[3:, :].set(0.0)
    cols, validf = pl.pallas_call(
        _select_kernel,
        grid=(_MPAD // _BQ,),
        in_specs=[
            pl.BlockSpec((_BQ, 3), lambda b: (b, 0)),
            pl.BlockSpec((8, _NPAD), lambda b: (0, 0)),
        ],
        out_specs=[
            pl.BlockSpec((_BQ, _K), lambda b: (b, 0)),
            pl.BlockSpec((_BQ, _K), lambda b: (b, 0)),
        ],
        out_shape=[
            jax.ShapeDtypeStruct((_MPAD, _K), jnp.int32),
            jax.ShapeDtypeStruct((_MPAD, _K), jnp.float32),
        ],
    )(qpadded, ptpad)

    nb = _MPAD // _BC
    colsr = cols.reshape(nb, _BC, _K).transpose(0, 2, 1).reshape(-1)
    vr = validf.reshape(nb, _BC, _K).transpose(0, 2, 1).reshape(-1, 1)
    xcat = jnp.concatenate([x, pos, jnp.zeros((_N, 5), jnp.float32)], axis=1)
    qpad8 = jnp.pad(pos_q, ((0, _MPAD - _M), (0, 5)))
    w1cat = jnp.concatenate([W1, jnp.zeros((5, 128), jnp.float32)], axis=0)
    w1p = jnp.concatenate([W1[128:131], jnp.zeros((5, 128), jnp.float32)], axis=0)
    out_pad = pl.pallas_call(
        _mlp_kernel,
        grid=(nb,),
        in_specs=[
            pl.BlockSpec((_E,), lambda b: (b,), memory_space=pltpu.SMEM),
            pl.BlockSpec((_E, 1), lambda b: (b, 0)),
            pl.BlockSpec((_N, 136), lambda b: (0, 0)),
            pl.BlockSpec((_BC, 8), lambda b: (b, 0)),
            pl.BlockSpec((136, 128), lambda b: (0, 0)),
            pl.BlockSpec((8, 128), lambda b: (0, 0)),
            pl.BlockSpec((1, 128), lambda b: (0, 0)),
            pl.BlockSpec((128, 128), lambda b: (0, 0)),
            pl.BlockSpec((1, 128), lambda b: (0, 0)),
        ],
        out_specs=pl.BlockSpec((_BC, 128), lambda b: (b, 0)),
        out_shape=jax.ShapeDtypeStruct((_MPAD, 128), jnp.float32),
        scratch_shapes=[pltpu.VMEM((_E, 136), jnp.float32)],
    )(colsr, vr, xcat, qpad8, w1cat, w1p, b1[None, :], W2, b2[None, :])
    out = out_pad[:_M]
    batch_out = jnp.take(batch, idx, axis=0)
    return (out, pos_q, batch_out)


# speedup vs baseline: 8.4458x; 1.0026x over previous
import functools

import jax
import jax.numpy as jnp
from jax.experimental import pallas as pl
from jax.experimental.pallas import tpu as pltpu

_RATIO = 0.5
_R = 0.2
_K = 32
_N = 10000
_NPAD = 10240
_ROWS = 8
_COLS = _NPAD // _ROWS
_M = int(_N * _RATIO)
_MPAD = 5120
_BC = 64
_E = _BC * _K
_BQ = 32


def _fold(ta, tb):
    va, na, xa, ya, za = ta
    vb, nb, xb, yb, zb = tb
    takeb = (vb > va) | ((vb == va) & (nb < na))
    return (
        jnp.where(takeb, vb, va),
        jnp.where(takeb, nb, na),
        jnp.where(takeb, xb, xa),
        jnp.where(takeb, yb, ya),
        jnp.where(takeb, zb, za),
    )


def _fps_kernel(px_ref, py_ref, pz_ref, idx_ref, *, n_samples):
    px = px_ref[...]
    py = py_ref[...]
    pz = pz_ref[...]
    n = (jax.lax.broadcasted_iota(jnp.int32, (_ROWS, _COLS), 0) * _COLS
         + jax.lax.broadcasted_iota(jnp.int32, (_ROWS, _COLS), 1))
    pad = n >= _N
    neg = jnp.float32(-jnp.inf)

    idx_ref[0] = jnp.int32(0)
    lx0 = px[0:1, 0:1]
    ly0 = py[0:1, 0:1]
    lz0 = pz[0:1, 0:1]
    dists0 = jnp.where(pad, neg, jnp.float32(jnp.inf))

    def argmax_tree(dists):
        parts = [
            tuple(a[:, i * 128 : (i + 1) * 128] for a in (dists, n, px, py, pz))
            for i in range(_COLS // 128)
        ]
        while len(parts) > 1:
            nxt = [
                _fold(parts[j], parts[j + 1]) for j in range(0, len(parts) - 1, 2)
            ]
            if len(parts) % 2:
                nxt.append(parts[-1])
            parts = nxt
        t = parts[0]
        r = 8
        while r > 1:
            h = r // 2
            t = _fold(tuple(a[:h, :] for a in t), tuple(a[h:, :] for a in t))
            r = h
        w = 128
        while w > 1:
            h = w // 2
            t = _fold(tuple(a[:, :h] for a in t), tuple(a[:, h:] for a in t))
            w = h
        return t

    def body(i, carry):
        lx, ly, lz, dists = carry
        dx = px - lx
        dy = py - ly
        dz = pz - lz
        d = (dx * dx + dz * dz) + dy * dy
        dists = jnp.minimum(dists, d)
        _, nn, xx, yy, zz = argmax_tree(dists)
        idx_ref[i] = jnp.min(nn)
        return (xx, yy, zz, dists)

    jax.lax.fori_loop(1, n_samples, body, (lx0, ly0, lz0, dists0))


def _select_kernel(q_ref, pt_ref, cols_ref, valid_ref):
    qv = q_ref[...]
    qpad = jnp.concatenate([qv, jnp.zeros((_BQ, 5), jnp.float32)], axis=1)
    pt = pt_ref[...]
    px = pt[0:1, :]
    py = pt[1:2, :]
    pz = pt[2:3, :]
    p2 = px * px + py * py + pz * pz
    qx = qv[:, 0:1]
    qy = qv[:, 1:2]
    qz = qv[:, 2:3]
    q2 = qx * qx + qy * qy + qz * qz
    qp = jnp.dot(qpad, pt, preferred_element_type=jnp.float32)
    d2 = q2 + p2 - 2.0 * qp
    r2 = jnp.float32(_R * _R)
    inf = jnp.float32(jnp.inf)
    bigi = jnp.int32(2**30)
    coln = jax.lax.broadcasted_iota(jnp.int32, (_BQ, _NPAD), 1)
    d2m = jnp.where(d2 <= r2, d2, inf)
    for k in range(_K):
        m = jnp.min(d2m, axis=1, keepdims=True)
        jv = jnp.min(jnp.where(d2m == m, coln, bigi), axis=1, keepdims=True)
        vk = m < inf
        cols_ref[:, k : k + 1] = jnp.where(vk, jv, 0)
        valid_ref[:, k : k + 1] = jnp.where(vk, 1.0, 0.0).astype(jnp.float32)
        d2m = jnp.where(coln == jv, inf, d2m)


def _mlp_kernel(colsr_ref, vr_ref, xcat_ref, q_ref, w1_ref, w1p_ref, b1_ref,
                w2_ref, b2_ref, o_ref, feat_ref):
    def gather_body(t, _):
        col = colsr_ref[t]
        feat_ref[pl.ds(t, 1), :] = xcat_ref[pl.ds(col, 1), :]
        return 0

    jax.lax.fori_loop(0, _E, gather_body, 0)
    f = feat_ref[...]
    h1 = jnp.dot(f, w1_ref[...], preferred_element_type=jnp.float32)
    corr = jnp.dot(q_ref[...], w1p_ref[...], preferred_element_type=jnp.float32)
    corrfull = jnp.concatenate([corr] * _K, axis=0)
    h1 = jnp.maximum(h1 - corrfull + b1_ref[...], 0.0)
    h2 = jnp.dot(h1, w2_ref[...], preferred_element_type=jnp.float32)
    h2 = jnp.maximum(h2 + b2_ref[...], 0.0)
    h2 = jnp.where(vr_ref[...] > 0.5, h2, -jnp.inf)
    acc = h2[0:_BC, :]
    for k in range(1, _K):
        acc = jnp.maximum(acc, h2[k * _BC : (k + 1) * _BC, :])
    o_ref[...] = acc


def kernel(x, pos, batch, W1, b1, W2, b2):
    pos = jax.lax.stop_gradient(pos)
    pn = jnp.pad(pos, ((0, _NPAD - _N), (0, 0)))
    px = pn[:, 0].reshape(_ROWS, _COLS)
    py = pn[:, 1].reshape(_ROWS, _COLS)
    pz = pn[:, 2].reshape(_ROWS, _COLS)
    idx = pl.pallas_call(
        functools.partial(_fps_kernel, n_samples=_M),
        out_shape=jax.ShapeDtypeStruct((_M,), jnp.int32),
        out_specs=pl.BlockSpec(memory_space=pltpu.SMEM),
    )(px, py, pz)
    pos_q = jnp.take(pos, idx, axis=0)

    qpadded = jnp.pad(pos_q, ((0, _MPAD - _M), (0, 0)), constant_values=1e9)
    ptpad = jnp.pad(pos.T, ((0, 5), (0, _NPAD - _N)), constant_values=-1e9)
    ptpad = ptpad.at[3:, :].set(0.0)
    cols, validf = pl.pallas_call(
        _select_kernel,
        grid=(_MPAD // _BQ,),
        in_specs=[
            pl.BlockSpec((_BQ, 3), lambda b: (b, 0)),
            pl.BlockSpec((8, _NPAD), lambda b: (0, 0)),
        ],
        out_specs=[
            pl.BlockSpec((_BQ, _K), lambda b: (b, 0)),
            pl.BlockSpec((_BQ, _K), lambda b: (b, 0)),
        ],
        out_shape=[
            jax.ShapeDtypeStruct((_MPAD, _K), jnp.int32),
            jax.ShapeDtypeStruct((_MPAD, _K), jnp.float32),
        ],
    )(qpadded, ptpad)

    nb = _MPAD // _BC
    colsr = cols.reshape(nb, _BC, _K).transpose(0, 2, 1).reshape(-1)
    vr = validf.reshape(nb, _BC, _K).transpose(0, 2, 1).reshape(-1, 1)
    xcat = jnp.concatenate([x, pos, jnp.zeros((_N, 5), jnp.float32)], axis=1)
    qpad8 = jnp.pad(pos_q, ((0, _MPAD - _M), (0, 5)))
    w1cat = jnp.concatenate([W1, jnp.zeros((5, 128), jnp.float32)], axis=0)
    w1p = jnp.concatenate([W1[128:131], jnp.zeros((5, 128), jnp.float32)], axis=0)
    out_pad = pl.pallas_call(
        _mlp_kernel,
        grid=(nb,),
        in_specs=[
            pl.BlockSpec((_E,), lambda b: (b,), memory_space=pltpu.SMEM),
            pl.BlockSpec((_E, 1), lambda b: (b, 0)),
            pl.BlockSpec((_N, 136), lambda b: (0, 0)),
            pl.BlockSpec((_BC, 8), lambda b: (b, 0)),
            pl.BlockSpec((136, 128), lambda b: (0, 0)),
            pl.BlockSpec((8, 128), lambda b: (0, 0)),
            pl.BlockSpec((1, 128), lambda b: (0, 0)),
            pl.BlockSpec((128, 128), lambda b: (0, 0)),
            pl.BlockSpec((1, 128), lambda b: (0, 0)),
        ],
        out_specs=pl.BlockSpec((_BC, 128), lambda b: (b, 0)),
        out_shape=jax.ShapeDtypeStruct((_MPAD, 128), jnp.float32),
        scratch_shapes=[pltpu.VMEM((_E, 136), jnp.float32)],
    )(colsr, vr, xcat, qpad8, w1cat, w1p, b1[None, :], W2, b2[None, :])
    out = out_pad[:_M]
    batch_out = jnp.take(batch, idx, axis=0)
    return (out, pos_q, batch_out)
